# baseline (speedup 1.0000x reference)
import jax
import jax.numpy as jnp
from jax.experimental import pallas as pl
from jax.experimental.pallas import tpu as pltpu


def _dma_copy_body(h_ref, o_ref, sem):
    pltpu.make_async_copy(h_ref, o_ref, sem).start()
    pltpu.make_async_copy(h_ref, o_ref, sem).wait()


def kernel(hidden_states, scores):
    B, T, D = hidden_states.shape
    out = pl.pallas_call(
        _dma_copy_body,
        in_specs=[pl.BlockSpec(memory_space=pl.ANY)],
        out_specs=pl.BlockSpec(memory_space=pl.ANY),
        out_shape=jax.ShapeDtypeStruct((B, T, D), hidden_states.dtype),
        scratch_shapes=[pltpu.SemaphoreType.DMA],
    )(hidden_states)
    return out


# speedup vs baseline: 36.2094x; 36.2094x over previous
import functools

import jax
import jax.numpy as jnp
from jax import lax
from jax.experimental import pallas as pl
from jax.experimental.pallas import tpu as pltpu
from jax.experimental.pallas import tpu_sc as plsc

NC, NS = 2, 16
NW = NC * NS
CH = 32
NBUF = 3


def _sc_copy_body(h_hbm, out_hbm, buf, sem_in, sem_out):
    wid = lax.axis_index("s") * NC + lax.axis_index("c")
    rows = h_hbm.shape[0] // NW
    base = wid * rows
    nch = rows // CH

    in_h = [None] * NBUF
    out_h = [None] * NBUF

    def load(i):
        slot = i % NBUF
        return pltpu.async_copy(
            h_hbm.at[pl.ds(base + i * CH, CH)], buf.at[slot], sem_in.at[slot])

    def store(i):
        slot = i % NBUF
        return pltpu.async_copy(
            buf.at[slot], out_hbm.at[pl.ds(base + i * CH, CH)], sem_out.at[slot])

    for i in range(nch):
        slot = i % NBUF
        if i >= NBUF:
            out_h[slot].wait()
        in_h[slot] = load(i)
        j = i - (NBUF - 1)
        if j >= 0:
            js = j % NBUF
            in_h[js].wait()
            out_h[js] = store(j)
    for j in range(max(0, nch - (NBUF - 1)), nch):
        js = j % NBUF
        in_h[js].wait()
        out_h[js] = store(j)
    for j in range(max(0, nch - NBUF), nch):
        out_h[j % NBUF].wait()


def kernel(hidden_states, scores):
    B, T, D = hidden_states.shape
    R = B * T
    h2 = hidden_states.reshape(R, D)
    mesh = plsc.VectorSubcoreMesh(core_axis_name="c", subcore_axis_name="s")
    copy_fn = pl.kernel(
        _sc_copy_body,
        out_type=jax.ShapeDtypeStruct((R, D), hidden_states.dtype),
        mesh=mesh,
        scratch_types=[
            pltpu.VMEM((NBUF, CH, D), hidden_states.dtype),
            pltpu.SemaphoreType.DMA((NBUF,)),
            pltpu.SemaphoreType.DMA((NBUF,)),
        ],
    )
    out = copy_fn(h2)
    return out.reshape(B, T, D)
